# baseline (speedup 1.0000x reference)
import dataclasses

import jax
import jax.numpy as jnp
from jax import lax
from jax.experimental import pallas as pl
from jax.experimental.pallas import tpu as pltpu
from jax.experimental.pallas import tpu_sc as plsc

NC = 2
NS = 16
CHUNK = 128


def _seg_sum_sc(vals, ed, with_count):
    _, n, d = vals.shape
    nchunks = ed.shape[0]
    nf = d // 16
    per = ((n // NS) // 8) * 8
    rem = n - NS * per
    base_chunks = nchunks // NS
    extra = nchunks - base_chunks * NS

    mesh = plsc.VectorSubcoreMesh(core_axis_name="c", subcore_axis_name="s")
    out_types = [jax.ShapeDtypeStruct((NC, n, d), jnp.float32)]
    scratch = [
        pltpu.VMEM_SHARED((n, d), jnp.float32),
        pltpu.VMEM((CHUNK, d), jnp.float32),
        pltpu.VMEM((CHUNK, d), jnp.float32),
        pltpu.VMEM((CHUNK, d), jnp.float32),
        pltpu.VMEM((3, CHUNK), jnp.int32),
        pltpu.VMEM((3, CHUNK), jnp.int32),
        pltpu.VMEM((3, CHUNK), jnp.int32),
        pltpu.VMEM((3, CHUNK), jnp.int32),
        pltpu.VMEM((3, CHUNK), jnp.int32),
        pltpu.VMEM((3, CHUNK), jnp.int32),
        pltpu.SemaphoreType.DMA,
        pltpu.SemaphoreType.DMA,
        pltpu.SemaphoreType.DMA,
        pltpu.SemaphoreType.DMA,
        pltpu.SemaphoreType.DMA,
        pltpu.SemaphoreType.DMA,
        pltpu.SemaphoreType.DMA,
        pltpu.SemaphoreType.DMA,
        pltpu.SemaphoreType.DMA,
        pltpu.SemaphoreType.DMA,
        pltpu.SemaphoreType.DMA,
        pltpu.SemaphoreType.DMA,
    ]
    if with_count:
        out_types.append(jax.ShapeDtypeStruct((n, 16), jnp.float32))
        scratch += [
            pltpu.VMEM_SHARED((n, 16), jnp.float32),
            pltpu.VMEM((CHUNK, 16), jnp.float32),
            pltpu.VMEM((CHUNK, 16), jnp.float32),
        ]

    def body(vals_h, ed_h, *refs):
        if with_count:
            (acc_h, cnt_h, *rest) = refs
            (acc_s, *rest), (cnt_s, ones_v, zc_v) = rest[:-3], rest[-3:]
        else:
            (acc_h, acc_s, *rest) = refs
        rbufs, rest = tuple(rest[:3]), rest[3:]
        ibufs, rest = tuple(rest[:6]), rest[6:]
        wbufs = (None,) * 6
        gsems, rest = tuple(rest[:3]), rest[3:]
        ssems, rest = tuple(rest[:3]), rest[3:]
        isems = tuple(rest[:6])
        r0 = rbufs[0]
        cid = lax.axis_index("c")
        sid = lax.axis_index("s")

        @pl.loop(0, CHUNK)
        def _(r):
            for t in range(nf):
                r0[r, pl.ds(t * 16, 16)] = jnp.zeros((16,), jnp.float32)

        nz, rz = per // CHUNK, per % CHUNK
        zbase = sid * per

        @pl.loop(0, nz)
        def _(j):
            pltpu.sync_copy(r0, acc_s.at[pl.ds(zbase + j * CHUNK, CHUNK)])
        if rz:
            pltpu.sync_copy(r0.at[pl.ds(0, rz)],
                            acc_s.at[pl.ds(zbase + nz * CHUNK, rz)])
        if rem:
            @pl.when(sid == NS - 1)
            def _():
                pltpu.sync_copy(r0.at[pl.ds(0, rem)],
                                acc_s.at[pl.ds(NS * per, rem)])

        if with_count:
            @pl.loop(0, CHUNK)
            def _(r):
                ones_v[r, pl.ds(0, 16)] = jnp.ones((16,), jnp.float32)
                zc_v[r, pl.ds(0, 16)] = jnp.zeros((16,), jnp.float32)

            @pl.when(cid == 0)
            def _():
                @pl.loop(0, nz)
                def _(j):
                    pltpu.sync_copy(zc_v,
                                    cnt_s.at[pl.ds(zbase + j * CHUNK, CHUNK)])
                if rz:
                    pltpu.sync_copy(zc_v.at[pl.ds(0, rz)],
                                    cnt_s.at[pl.ds(zbase + nz * CHUNK, rz)])
                if rem:
                    @pl.when(sid == NS - 1)
                    def _():
                        pltpu.sync_copy(zc_v.at[pl.ds(0, rem)],
                                        cnt_s.at[pl.ds(NS * per, rem)])

        plsc.subcore_barrier()

        def gather(ibuf, rbuf, gsem):
            pltpu.async_copy(vals_h.at[cid].at[ibuf.at[0]], rbuf, gsem)

        def wait_g(rbuf, gsem):
            pltpu.make_async_copy(vals_h.at[cid].at[pl.ds(0, CHUNK)],
                                  rbuf, gsem).wait()

        row2 = jnp.full((16,), 2, jnp.int32)

        def fetch_idx(c, ibuf, wbuf, isem):
            del wbuf
            pltpu.async_copy(ed_h.at[c], ibuf, isem)

        def wait_i(ibuf, wbuf, isem):
            del wbuf
            pltpu.make_async_copy(ed_h.at[0], ibuf, isem).wait()

        lane_dnums = lax.GatherDimensionNumbers(
            offset_dims=(), collapsed_slice_dims=(0,), start_index_map=(0,))

        def multiply(rbuf, ibuf):
            @plsc.parallel_loop(0, CHUNK // 16, unroll=2)
            def _(gi):
                wvec = plsc.bitcast(ibuf[2, pl.ds(gi * 16, 16)], jnp.float32)
                for e in range(16):
                    ws = lax.gather(
                        wvec, jnp.full((16, 1), e, jnp.int32), lane_dnums,
                        (1,), mode=lax.GatherScatterMode.PROMISE_IN_BOUNDS)
                    ei = gi * 16 + e
                    for t in range(nf):
                        sl = pl.ds(t * 16, 16)
                        rbuf[ei, sl] = rbuf[ei, sl] * ws

        def scatter_async(rbuf, ibuf, ssem):
            pltpu.async_copy(rbuf, acc_s.at[ibuf.at[1]], ssem, add=True)
            if with_count:
                @pl.when(cid == 0)
                def _():
                    pltpu.async_copy(ones_v, cnt_s.at[ibuf.at[1]], ssem,
                                     add=True)

        def wait_s(rbuf, ssem):
            pltpu.make_async_copy(vals_h.at[cid].at[pl.ds(0, CHUNK)],
                                  rbuf, ssem).wait()
            if with_count:
                @pl.when(cid == 0)
                def _():
                    pltpu.make_async_copy(cnt_h.at[pl.ds(0, CHUNK)],
                                          ones_v, ssem).wait()

        c0 = sid * base_chunks
        last = base_chunks - 1
        nsext = base_chunks // 6
        fetch_idx(c0, ibufs[0], wbufs[0], isems[0])
        fetch_idx(c0 + 1, ibufs[1], wbufs[1], isems[1])
        fetch_idx(c0 + 2, ibufs[2], wbufs[2], isems[2])
        wait_i(ibufs[0], wbufs[0], isems[0])
        gather(ibufs[0], rbufs[0], gsems[0])

        @pl.loop(0, nsext)
        def _(j):
            for p in range(6):
                q3, q6 = p % 3, p
                wait_g(rbufs[q3], gsems[q3])

                def w(q3=q3):
                    wait_s(rbufs[(q3 + 1) % 3], ssems[(q3 + 1) % 3])
                if p < 2:
                    pl.when(j > 0)(w)
                else:
                    w()

                def g(q3=q3, q6=q6):
                    wait_i(ibufs[(q6 + 1) % 6], wbufs[(q6 + 1) % 6],
                           isems[(q6 + 1) % 6])
                    gather(ibufs[(q6 + 1) % 6], rbufs[(q3 + 1) % 3],
                           gsems[(q3 + 1) % 3])
                if p == 5:
                    pl.when(j < nsext - 1)(g)
                else:
                    g()

                def f(j=j, p=p, q6=q6):
                    fetch_idx(c0 + 6 * j + p + 3, ibufs[(q6 + 3) % 6],
                              wbufs[(q6 + 3) % 6], isems[(q6 + 3) % 6])
                if p < 3:
                    f()
                else:
                    pl.when(j < nsext - 1)(f)

                multiply(rbufs[q3], ibufs[q6])
                scatter_async(rbufs[q3], ibufs[q6], ssems[q3])

        wait_s(rbufs[(last - 1) % 3], ssems[(last - 1) % 3])
        wait_s(rbufs[last % 3], ssems[last % 3])

        if extra:
            @pl.when(sid < extra)
            def _():
                ce = NS * base_chunks + sid
                pltpu.sync_copy(ed_h.at[ce], ibufs[0])
                gather(ibufs[0], rbufs[0], gsems[0])
                wait_g(rbufs[0], gsems[0])
                multiply(rbufs[0], ibufs[0])
                scatter_async(rbufs[0], ibufs[0], ssems[0])
                wait_s(rbufs[0], ssems[0])

        plsc.subcore_barrier()

        pltpu.sync_copy(acc_s.at[pl.ds(zbase, per)],
                        acc_h.at[cid, pl.ds(zbase, per)])
        if rem:
            @pl.when(sid == NS - 1)
            def _():
                pltpu.sync_copy(acc_s.at[pl.ds(NS * per, rem)],
                                acc_h.at[cid, pl.ds(NS * per, rem)])
        if with_count:
            @pl.when(cid == 0)
            def _():
                pltpu.sync_copy(cnt_s.at[pl.ds(zbase, per)],
                                cnt_h.at[pl.ds(zbase, per)])
                if rem:
                    @pl.when(sid == NS - 1)
                    def _():
                        pltpu.sync_copy(cnt_s.at[pl.ds(NS * per, rem)],
                                        cnt_h.at[pl.ds(NS * per, rem)])

    cp = pltpu.CompilerParams()
    for f, v in (("needs_layout_passes", False),
                 ("use_tc_tiling_on_sc", False)):
        if f in pltpu.CompilerParams.__dataclass_fields__:
            cp = dataclasses.replace(cp, **{f: v})
    fn = pl.kernel(body, out_type=out_types, mesh=mesh, scratch_types=scratch,
                   compiler_params=cp)
    out = fn(vals, ed)
    return (out[0], out[1]) if with_count else out[0]


def _tc_layer0(accp, cnt16, x, wl0, wr0, wl1, wr1, b0, b1):
    n, d_in = x.shape
    d_out = wl1.shape[1]
    half = d_out // 2
    blk = 1000

    def body(accp_r, cnt_r, x_r, wl0_r, wr0_r, wl1_r, wr1_r, b0_r, b1_r,
             y1_r, outr_r):
        acc = jnp.concatenate([accp_r[0], accp_r[1]], axis=-1)
        cnt = cnt_r[:, 0:1]
        mean = acc / jnp.maximum(cnt, 1.0)
        h = jnp.maximum(
            jnp.dot(mean, wl0_r[...], preferred_element_type=jnp.float32)
            + jnp.dot(x_r[...], wr0_r[...], preferred_element_type=jnp.float32)
            + b0_r[...], 0.0)
        y1 = jnp.dot(h, wl1_r[...], preferred_element_type=jnp.float32)
        y1_r[0] = y1[:, :half]
        y1_r[1] = y1[:, half:]
        outr_r[...] = (jnp.dot(h, wr1_r[...], preferred_element_type=jnp.float32)
                       + b1_r[...])

    return pl.pallas_call(
        body,
        grid=(n // blk,),
        in_specs=[
            pl.BlockSpec((NC, blk, d_in // 2), lambda i: (0, i, 0)),
            pl.BlockSpec((blk, 16), lambda i: (i, 0)),
            pl.BlockSpec((blk, d_in), lambda i: (i, 0)),
            pl.BlockSpec((d_in, d_in), lambda i: (0, 0)),
            pl.BlockSpec((d_in, d_in), lambda i: (0, 0)),
            pl.BlockSpec((d_in, d_out), lambda i: (0, 0)),
            pl.BlockSpec((d_in, d_out), lambda i: (0, 0)),
            pl.BlockSpec((1, d_in), lambda i: (0, 0)),
            pl.BlockSpec((1, d_out), lambda i: (0, 0)),
        ],
        out_specs=[
            pl.BlockSpec((NC, blk, half), lambda i: (0, i, 0)),
            pl.BlockSpec((blk, d_out), lambda i: (i, 0)),
        ],
        out_shape=[
            jax.ShapeDtypeStruct((NC, n, half), jnp.float32),
            jax.ShapeDtypeStruct((n, d_out), jnp.float32),
        ],
    )(accp, cnt16, x, wl0, wr0, wl1, wr1, b0, b1)


def _tc_final(acc2p, cnt16, outr):
    n, d_out = outr.shape
    blk = 1000

    def body(acc2p_r, cnt_r, outr_r, o_r):
        a = jnp.concatenate([acc2p_r[0], acc2p_r[1]], axis=-1)
        cnt = cnt_r[:, 0:1]
        logits = a / jnp.maximum(cnt, 1.0) + outr_r[...]
        m = jnp.max(logits, axis=-1, keepdims=True)
        lse = m + jnp.log(jnp.sum(jnp.exp(logits - m), axis=-1, keepdims=True))
        o_r[...] = logits - lse

    return pl.pallas_call(
        body,
        grid=(n // blk,),
        in_specs=[
            pl.BlockSpec((NC, blk, d_out // 2), lambda i: (0, i, 0)),
            pl.BlockSpec((blk, 16), lambda i: (i, 0)),
            pl.BlockSpec((blk, d_out), lambda i: (i, 0)),
        ],
        out_specs=pl.BlockSpec((blk, d_out), lambda i: (i, 0)),
        out_shape=jax.ShapeDtypeStruct((n, d_out), jnp.float32),
    )(acc2p, cnt16, outr)


def kernel(x, edge_index, edge_attr, W_l0, b_l0, W_r0, b_r0,
           W_l1, b_l1, W_r1, b_r1):
    n, d_in = x.shape
    e = edge_attr.shape[0]
    b0 = (b_l0 + b_r0)[None, :]
    b1 = (b_l1 + b_r1)[None, :]
    xs = jnp.moveaxis(x.reshape(n, NC, d_in // NC), 1, 0)
    wbits = jax.lax.bitcast_convert_type(edge_attr, jnp.int32)
    ed = (jnp.stack([edge_index[0], edge_index[1], wbits], axis=0)
          .reshape(3, e // CHUNK, CHUNK).transpose(1, 0, 2))
    acc0, cnt16 = _seg_sum_sc(xs, ed, with_count=True)
    y1, outr = _tc_layer0(acc0, cnt16, x, W_l0, W_r0, W_l1, W_r1, b0, b1)
    acc1 = _seg_sum_sc(y1, ed, with_count=False)
    return _tc_final(acc1, cnt16, outr)


# speedup vs baseline: 1.2144x; 1.2144x over previous
import dataclasses

import jax
import jax.numpy as jnp
from jax import lax
from jax.experimental import pallas as pl
from jax.experimental.pallas import tpu as pltpu
from jax.experimental.pallas import tpu_sc as plsc

NC = 2
NS = 16
CHUNK = 128


def _seg_sum_sc(vals, ed, with_count):
    _, n, d = vals.shape
    npairs = ed.shape[0] // 2
    nf = d // 16
    per = ((n // NS) // 8) * 8
    rem = n - NS * per
    base_chunks = npairs // NS
    extra = npairs - base_chunks * NS

    mesh = plsc.VectorSubcoreMesh(core_axis_name="c", subcore_axis_name="s")
    out_types = [jax.ShapeDtypeStruct((NC, n, d), jnp.float32)]
    scratch = [
        pltpu.VMEM_SHARED((n, d), jnp.float32),
        pltpu.VMEM((2 * CHUNK, d), jnp.float32),
        pltpu.VMEM((2 * CHUNK, d), jnp.float32),
        pltpu.VMEM((2 * CHUNK, d), jnp.float32),
        pltpu.VMEM((2, 3, CHUNK), jnp.int32),
        pltpu.VMEM((2, 3, CHUNK), jnp.int32),
        pltpu.VMEM((2, 3, CHUNK), jnp.int32),
        pltpu.VMEM((2, 3, CHUNK), jnp.int32),
        pltpu.VMEM((2, 3, CHUNK), jnp.int32),
        pltpu.VMEM((2, 3, CHUNK), jnp.int32),
        pltpu.SemaphoreType.DMA,
        pltpu.SemaphoreType.DMA,
        pltpu.SemaphoreType.DMA,
        pltpu.SemaphoreType.DMA,
        pltpu.SemaphoreType.DMA,
        pltpu.SemaphoreType.DMA,
        pltpu.SemaphoreType.DMA,
        pltpu.SemaphoreType.DMA,
        pltpu.SemaphoreType.DMA,
        pltpu.SemaphoreType.DMA,
        pltpu.SemaphoreType.DMA,
        pltpu.SemaphoreType.DMA,
    ]
    if with_count:
        out_types.append(jax.ShapeDtypeStruct((n, 16), jnp.float32))
        scratch += [
            pltpu.VMEM_SHARED((n, 16), jnp.float32),
            pltpu.VMEM((CHUNK, 16), jnp.float32),
            pltpu.VMEM((CHUNK, 16), jnp.float32),
        ]

    def body(vals_h, ed_h, *refs):
        if with_count:
            (acc_h, cnt_h, *rest) = refs
            (acc_s, *rest), (cnt_s, ones_v, zc_v) = rest[:-3], rest[-3:]
        else:
            (acc_h, acc_s, *rest) = refs
        rbufs, rest = tuple(rest[:3]), rest[3:]
        ibufs, rest = tuple(rest[:6]), rest[6:]
        wbufs = (None,) * 6
        gsems, rest = tuple(rest[:3]), rest[3:]
        ssems, rest = tuple(rest[:3]), rest[3:]
        isems = tuple(rest[:6])
        r0 = rbufs[0]
        cid = lax.axis_index("c")
        sid = lax.axis_index("s")

        ZR = 2 * CHUNK

        @pl.loop(0, ZR)
        def _(r):
            for t in range(nf):
                r0[r, pl.ds(t * 16, 16)] = jnp.zeros((16,), jnp.float32)

        nz, rz = per // ZR, per % ZR
        zbase = sid * per

        @pl.loop(0, nz)
        def _(j):
            pltpu.sync_copy(r0, acc_s.at[pl.ds(zbase + j * ZR, ZR)])
        if rz:
            pltpu.sync_copy(r0.at[pl.ds(0, rz)],
                            acc_s.at[pl.ds(zbase + nz * ZR, rz)])
        if rem:
            @pl.when(sid == NS - 1)
            def _():
                pltpu.sync_copy(r0.at[pl.ds(0, rem)],
                                acc_s.at[pl.ds(NS * per, rem)])

        if with_count:
            @pl.loop(0, CHUNK)
            def _(r):
                ones_v[r, pl.ds(0, 16)] = jnp.ones((16,), jnp.float32)
                zc_v[r, pl.ds(0, 16)] = jnp.zeros((16,), jnp.float32)

            nz2, rz2 = per // CHUNK, per % CHUNK

            @pl.when(cid == 0)
            def _():
                @pl.loop(0, nz2)
                def _(j):
                    pltpu.sync_copy(zc_v,
                                    cnt_s.at[pl.ds(zbase + j * CHUNK, CHUNK)])
                if rz2:
                    pltpu.sync_copy(zc_v.at[pl.ds(0, rz2)],
                                    cnt_s.at[pl.ds(zbase + nz2 * CHUNK, rz2)])
                if rem:
                    @pl.when(sid == NS - 1)
                    def _():
                        pltpu.sync_copy(zc_v.at[pl.ds(0, rem)],
                                        cnt_s.at[pl.ds(NS * per, rem)])

        plsc.subcore_barrier()

        def gather(ibuf, rbuf, gsem):
            pltpu.async_copy(vals_h.at[cid].at[ibuf.at[0, 0]],
                             rbuf.at[pl.ds(0, CHUNK)], gsem)
            pltpu.async_copy(vals_h.at[cid].at[ibuf.at[1, 0]],
                             rbuf.at[pl.ds(CHUNK, CHUNK)], gsem)

        def wait_g(rbuf, gsem):
            pltpu.make_async_copy(vals_h.at[cid].at[pl.ds(0, 2 * CHUNK)],
                                  rbuf, gsem).wait()

        row2 = jnp.full((16,), 2, jnp.int32)

        def fetch_idx(c, ibuf, isem):
            pltpu.async_copy(ed_h.at[pl.ds(2 * c, 2)], ibuf, isem)

        def wait_i(ibuf, isem):
            pltpu.make_async_copy(ed_h.at[pl.ds(0, 2)], ibuf, isem).wait()

        def multiply(rbuf, ibuf):
            for h in range(2):
                ib_h = ibuf.at[h]

                @plsc.parallel_loop(0, CHUNK, unroll=4)
                def _(ei, ib_h=ib_h, base=h * CHUNK):
                    wraw = plsc.load_gather(
                        ib_h, [row2, lax.broadcast_in_dim(ei, (16,), ())])
                    ws = plsc.bitcast(wraw, jnp.float32)
                    for t in range(nf):
                        sl = pl.ds(t * 16, 16)
                        rbuf[base + ei, sl] = rbuf[base + ei, sl] * ws

        def scatter_async(rbuf, ibuf, ssem):
            pltpu.async_copy(rbuf.at[pl.ds(0, CHUNK)],
                             acc_s.at[ibuf.at[0, 1]], ssem, add=True)
            pltpu.async_copy(rbuf.at[pl.ds(CHUNK, CHUNK)],
                             acc_s.at[ibuf.at[1, 1]], ssem, add=True)
            if with_count:
                @pl.when(cid == 0)
                def _():
                    pltpu.async_copy(ones_v, cnt_s.at[ibuf.at[0, 1]], ssem,
                                     add=True)
                    pltpu.async_copy(ones_v, cnt_s.at[ibuf.at[1, 1]], ssem,
                                     add=True)

        def wait_s(rbuf, ssem):
            pltpu.make_async_copy(vals_h.at[cid].at[pl.ds(0, 2 * CHUNK)],
                                  rbuf, ssem).wait()
            if with_count:
                @pl.when(cid == 0)
                def _():
                    for _ in range(2):
                        pltpu.make_async_copy(cnt_h.at[pl.ds(0, CHUNK)],
                                              ones_v, ssem).wait()

        c0 = sid * base_chunks
        last = base_chunks - 1
        nsext = base_chunks // 6
        fetch_idx(c0, ibufs[0], isems[0])
        fetch_idx(c0 + 1, ibufs[1], isems[1])
        fetch_idx(c0 + 2, ibufs[2], isems[2])
        wait_i(ibufs[0], isems[0])
        gather(ibufs[0], rbufs[0], gsems[0])

        @pl.loop(0, nsext)
        def _(j):
            for p in range(6):
                q3, q6 = p % 3, p
                wait_g(rbufs[q3], gsems[q3])

                def w(q3=q3):
                    wait_s(rbufs[(q3 + 1) % 3], ssems[(q3 + 1) % 3])
                if p < 2:
                    pl.when(j > 0)(w)
                else:
                    w()

                def g(q3=q3, q6=q6):
                    wait_i(ibufs[(q6 + 1) % 6], isems[(q6 + 1) % 6])
                    gather(ibufs[(q6 + 1) % 6], rbufs[(q3 + 1) % 3],
                           gsems[(q3 + 1) % 3])
                if p == 5:
                    pl.when(j < nsext - 1)(g)
                else:
                    g()

                def f(j=j, p=p, q6=q6):
                    fetch_idx(c0 + 6 * j + p + 3, ibufs[(q6 + 3) % 6],
                              isems[(q6 + 3) % 6])
                if p < 3:
                    f()
                else:
                    pl.when(j < nsext - 1)(f)

                multiply(rbufs[q3], ibufs[q6])
                scatter_async(rbufs[q3], ibufs[q6], ssems[q3])

        wait_s(rbufs[(last - 1) % 3], ssems[(last - 1) % 3])
        wait_s(rbufs[last % 3], ssems[last % 3])

        if extra:
            @pl.when(sid < extra)
            def _():
                ce = NS * base_chunks + sid
                pltpu.sync_copy(ed_h.at[pl.ds(2 * ce, 2)], ibufs[0])
                gather(ibufs[0], rbufs[0], gsems[0])
                wait_g(rbufs[0], gsems[0])
                multiply(rbufs[0], ibufs[0])
                scatter_async(rbufs[0], ibufs[0], ssems[0])
                wait_s(rbufs[0], ssems[0])

        plsc.subcore_barrier()

        pltpu.sync_copy(acc_s.at[pl.ds(zbase, per)],
                        acc_h.at[cid, pl.ds(zbase, per)])
        if rem:
            @pl.when(sid == NS - 1)
            def _():
                pltpu.sync_copy(acc_s.at[pl.ds(NS * per, rem)],
                                acc_h.at[cid, pl.ds(NS * per, rem)])
        if with_count:
            @pl.when(cid == 0)
            def _():
                pltpu.sync_copy(cnt_s.at[pl.ds(zbase, per)],
                                cnt_h.at[pl.ds(zbase, per)])
                if rem:
                    @pl.when(sid == NS - 1)
                    def _():
                        pltpu.sync_copy(cnt_s.at[pl.ds(NS * per, rem)],
                                        cnt_h.at[pl.ds(NS * per, rem)])

    cp = pltpu.CompilerParams()
    for f, v in (("needs_layout_passes", False),
                 ("use_tc_tiling_on_sc", False)):
        if f in pltpu.CompilerParams.__dataclass_fields__:
            cp = dataclasses.replace(cp, **{f: v})
    fn = pl.kernel(body, out_type=out_types, mesh=mesh, scratch_types=scratch,
                   compiler_params=cp)
    out = fn(vals, ed)
    return (out[0], out[1]) if with_count else out[0]


def _tc_layer0(accp, cnt16, x, wl0, wr0, wl1, wr1, b0, b1):
    n, d_in = x.shape
    d_out = wl1.shape[1]
    half = d_out // 2
    blk = 1000

    def body(accp_r, cnt_r, x_r, wl0_r, wr0_r, wl1_r, wr1_r, b0_r, b1_r,
             y1_r, outr_r):
        acc = jnp.concatenate([accp_r[0], accp_r[1]], axis=-1)
        cnt = cnt_r[:, 0:1]
        mean = acc / jnp.maximum(cnt, 1.0)
        h = jnp.maximum(
            jnp.dot(mean, wl0_r[...], preferred_element_type=jnp.float32)
            + jnp.dot(x_r[...], wr0_r[...], preferred_element_type=jnp.float32)
            + b0_r[...], 0.0)
        y1 = jnp.dot(h, wl1_r[...], preferred_element_type=jnp.float32)
        y1_r[0] = y1[:, :half]
        y1_r[1] = y1[:, half:]
        outr_r[...] = (jnp.dot(h, wr1_r[...], preferred_element_type=jnp.float32)
                       + b1_r[...])

    return pl.pallas_call(
        body,
        grid=(n // blk,),
        in_specs=[
            pl.BlockSpec((NC, blk, d_in // 2), lambda i: (0, i, 0)),
            pl.BlockSpec((blk, 16), lambda i: (i, 0)),
            pl.BlockSpec((blk, d_in), lambda i: (i, 0)),
            pl.BlockSpec((d_in, d_in), lambda i: (0, 0)),
            pl.BlockSpec((d_in, d_in), lambda i: (0, 0)),
            pl.BlockSpec((d_in, d_out), lambda i: (0, 0)),
            pl.BlockSpec((d_in, d_out), lambda i: (0, 0)),
            pl.BlockSpec((1, d_in), lambda i: (0, 0)),
            pl.BlockSpec((1, d_out), lambda i: (0, 0)),
        ],
        out_specs=[
            pl.BlockSpec((NC, blk, half), lambda i: (0, i, 0)),
            pl.BlockSpec((blk, d_out), lambda i: (i, 0)),
        ],
        out_shape=[
            jax.ShapeDtypeStruct((NC, n, half), jnp.float32),
            jax.ShapeDtypeStruct((n, d_out), jnp.float32),
        ],
    )(accp, cnt16, x, wl0, wr0, wl1, wr1, b0, b1)


def _tc_final(acc2p, cnt16, outr):
    n, d_out = outr.shape
    blk = 1000

    def body(acc2p_r, cnt_r, outr_r, o_r):
        a = jnp.concatenate([acc2p_r[0], acc2p_r[1]], axis=-1)
        cnt = cnt_r[:, 0:1]
        logits = a / jnp.maximum(cnt, 1.0) + outr_r[...]
        m = jnp.max(logits, axis=-1, keepdims=True)
        lse = m + jnp.log(jnp.sum(jnp.exp(logits - m), axis=-1, keepdims=True))
        o_r[...] = logits - lse

    return pl.pallas_call(
        body,
        grid=(n // blk,),
        in_specs=[
            pl.BlockSpec((NC, blk, d_out // 2), lambda i: (0, i, 0)),
            pl.BlockSpec((blk, 16), lambda i: (i, 0)),
            pl.BlockSpec((blk, d_out), lambda i: (i, 0)),
        ],
        out_specs=pl.BlockSpec((blk, d_out), lambda i: (i, 0)),
        out_shape=jax.ShapeDtypeStruct((n, d_out), jnp.float32),
    )(acc2p, cnt16, outr)


def kernel(x, edge_index, edge_attr, W_l0, b_l0, W_r0, b_r0,
           W_l1, b_l1, W_r1, b_r1):
    n, d_in = x.shape
    e = edge_attr.shape[0]
    b0 = (b_l0 + b_r0)[None, :]
    b1 = (b_l1 + b_r1)[None, :]
    xs = jnp.moveaxis(x.reshape(n, NC, d_in // NC), 1, 0)
    wbits = jax.lax.bitcast_convert_type(edge_attr, jnp.int32)
    ed = (jnp.stack([edge_index[0], edge_index[1], wbits], axis=0)
          .reshape(3, e // CHUNK, CHUNK).transpose(1, 0, 2))
    acc0, cnt16 = _seg_sum_sc(xs, ed, with_count=True)
    y1, outr = _tc_layer0(acc0, cnt16, x, W_l0, W_r0, W_l1, W_r1, b0, b1)
    acc1 = _seg_sum_sc(y1, ed, with_count=False)
    return _tc_final(acc1, cnt16, outr)
